# baseline (speedup 1.0000x reference)
import functools

import jax
import jax.numpy as jnp
from jax import lax
from jax.experimental import pallas as pl
from jax.experimental.pallas import tpu as pltpu
from jax.experimental.pallas import tpu_sc as plsc

NC, NS = 2, 16
NW = NC * NS
CHUNK = 128
ROWS_PER_TILE = 640
N_PAD = NS * ROWS_PER_TILE
DEG_W = 16


def _zero_acc_slice(zbuf, acc, sid, d, sem):
    zero = jnp.zeros((16,), jnp.float32)

    def zfill(i, _):
        zbuf[i // (d // 16), pl.ds((i % (d // 16)) * 16, 16)] = zero
        return 0

    lax.fori_loop(0, 16 * (d // 16), zfill, 0)
    base = sid * ROWS_PER_TILE
    nz = ROWS_PER_TILE // 16
    descs = [pltpu.async_copy(zbuf, acc.at[pl.ds(base + r * 16, 16)], sem)
             for r in range(nz)]
    for dsc in descs:
        dsc.wait()


SLOW_CID = 1


def _agg_pipeline(tbl, acc, sidx, didx, rows, semg, sems, k, GB):
    assert k % (2 * GB) == 0

    def gath(j, b):
        return pltpu.async_copy(tbl.at[sidx.at[j]], rows.at[b], semg)

    def gath_wait(j, b):
        pltpu.make_async_copy(tbl.at[sidx.at[j]], rows.at[b], semg).wait()

    def scat(j, b):
        return pltpu.async_copy(rows.at[b], acc.at[didx.at[j]], sems,
                                add=True)

    for i in range(GB):
        gath(i, i)

    def step(g, _):
        j0 = 2 * GB * g
        for p in range(2):
            ja = j0 + p * GB
            jn = ja + GB
            for i in range(GB):
                gath_wait(ja + i, p * GB + i)

            @pl.when(jn < k)
            def _():
                for i in range(GB):
                    gath(jn + i, (1 - p) * GB + i)

            sc = [scat(ja + i, p * GB + i) for i in range(GB)]
            for dsc in sc:
                dsc.wait()
        return 0

    lax.fori_loop(0, k // (2 * GB), step, 0)


@functools.lru_cache(maxsize=None)
def _make_agg(d, ks, kf):
    mesh = plsc.VectorSubcoreMesh(core_axis_name="c", subcore_axis_name="s")

    def body(tbl, srcp, dstp, out, sidx, didx, rows, zbuf, acc, semg, sems):
        cid = lax.axis_index("c")
        sid = lax.axis_index("s")
        _zero_acc_slice(zbuf, acc, sid, d, sems)
        plsc.subcore_barrier()

        def run(base, kc, gb):
            pltpu.sync_copy(srcp.at[pl.ds(base, kc)], sidx.at[pl.ds(0, kc)])
            pltpu.sync_copy(dstp.at[pl.ds(base, kc)], didx.at[pl.ds(0, kc)])
            _agg_pipeline(tbl, acc, sidx, didx, rows, semg, sems, kc, gb)

        @pl.when(cid == SLOW_CID)
        def _():
            run(sid * ks, ks, 2)

        @pl.when(cid != SLOW_CID)
        def _():
            run(NS * ks + sid * kf, kf, 3)

        plsc.subcore_barrier()
        base = sid * ROWS_PER_TILE
        pltpu.sync_copy(acc.at[pl.ds(base, ROWS_PER_TILE)],
                        out.at[cid, pl.ds(base, ROWS_PER_TILE)])

    return pl.kernel(
        body,
        out_type=jax.ShapeDtypeStruct((NC, N_PAD, d), jnp.float32),
        mesh=mesh,
        scratch_types=[
            pltpu.VMEM((max(ks, kf), CHUNK), jnp.int32),
            pltpu.VMEM((max(ks, kf), CHUNK), jnp.int32),
            pltpu.VMEM((6, CHUNK, d), jnp.float32),
            pltpu.VMEM((16, d), jnp.float32),
            pltpu.VMEM_SHARED((N_PAD, d), jnp.float32),
            pltpu.SemaphoreType.DMA,
            pltpu.SemaphoreType.DMA,
        ],
        compiler_params=pltpu.CompilerParams(use_tc_tiling_on_sc=False),
    )


@functools.lru_cache(maxsize=None)
def _make_deg(ks, kf):
    mesh = plsc.VectorSubcoreMesh(core_axis_name="c", subcore_axis_name="s")

    GB = 8
    assert ks % GB == 0 and kf % GB == 0

    def body(dstp, out, didx, ones_rows, zbuf, acc, sems):
        cid = lax.axis_index("c")
        sid = lax.axis_index("s")
        one = jnp.full((16,), 1.0, jnp.float32)

        def ofill(r, _):
            ones_rows[r, pl.ds(0, 16)] = one
            return 0

        lax.fori_loop(0, CHUNK, ofill, 0)
        _zero_acc_slice(zbuf, acc, sid, DEG_W, sems)
        plsc.subcore_barrier()

        def run(base, kc):
            pltpu.sync_copy(dstp.at[pl.ds(base, kc)], didx.at[pl.ds(0, kc)])

            def step(g, _):
                j0 = GB * g
                sc = [pltpu.async_copy(ones_rows, acc.at[didx.at[j0 + i]],
                                       sems, add=True) for i in range(GB)]
                for dsc in sc:
                    dsc.wait()
                return 0

            lax.fori_loop(0, kc // GB, step, 0)

        @pl.when(cid == SLOW_CID)
        def _():
            run(sid * ks, ks)

        @pl.when(cid != SLOW_CID)
        def _():
            run(NS * ks + sid * kf, kf)

        plsc.subcore_barrier()
        base = sid * ROWS_PER_TILE
        pltpu.sync_copy(acc.at[pl.ds(base, ROWS_PER_TILE)],
                        out.at[cid, pl.ds(base, ROWS_PER_TILE)])

    return pl.kernel(
        body,
        out_type=jax.ShapeDtypeStruct((NC, N_PAD, DEG_W), jnp.float32),
        mesh=mesh,
        scratch_types=[
            pltpu.VMEM((max(ks, kf), CHUNK), jnp.int32),
            pltpu.VMEM((CHUNK, DEG_W), jnp.float32),
            pltpu.VMEM((16, DEG_W), jnp.float32),
            pltpu.VMEM_SHARED((N_PAD, DEG_W), jnp.float32),
            pltpu.SemaphoreType.DMA,
        ],
        compiler_params=pltpu.CompilerParams(use_tc_tiling_on_sc=False),
    )


BR = 400


def _dis_block(degp):
    p = degp[0, :, :1] + degp[1, :, :1]
    return lax.rsqrt(p + 1.0)


def _tc1_body(degp, x, w1, y1):
    dis = _dis_block(degp)
    y1[...] = dis * jnp.dot(x[...], w1[...], preferred_element_type=jnp.float32)


def _tc2_body(degp, aggp, y1, w2, b1, y2):
    dis = _dis_block(degp)
    h = jnp.maximum(dis * (aggp[0] + aggp[1] + y1[...]) + b1[...], 0.0)
    y2[...] = dis * jnp.dot(h, w2[...], preferred_element_type=jnp.float32)


def _tc3_body(degp, aggp, y2, b2, wfc, bfc, out):
    dis = _dis_block(degp)
    h = jnp.maximum(dis * (aggp[0] + aggp[1] + y2[...]) + b2[...], 0.0)
    logit = jnp.dot(h, wfc[...], preferred_element_type=jnp.float32) + bfc[...]
    m = jnp.max(logit, axis=1, keepdims=True)
    s = jnp.sum(jnp.exp(logit - m), axis=1, keepdims=True)
    out[...] = (logit - m) - jnp.log(s)


def _degp_spec():
    return pl.BlockSpec((NC, BR, DEG_W), lambda i: (0, i, 0))


def _aggp_spec(d):
    return pl.BlockSpec((NC, BR, d), lambda i: (0, i, 0))


def _full(shape):
    nd = len(shape)
    return pl.BlockSpec(shape, lambda i: (0,) * nd)


def kernel(x, edge_index, W1, b1, W2, b2, Wfc, bfc):
    n = x.shape[0]
    e = edge_index.shape[1]
    f_in, d1 = W1.shape
    d2 = W2.shape[1]
    c = Wfc.shape[1]
    assert n % BR == 0

    src = edge_index[0].astype(jnp.int32)
    dst = edge_index[1].astype(jnp.int32)
    k = -(-e // (NW * CHUNK))
    k = -(-k // 8) * 8
    e_pad = NW * k * CHUNK
    srcp = jnp.concatenate([src, jnp.zeros((e_pad - e,), jnp.int32)])
    dstp = jnp.concatenate([dst, jnp.full((e_pad - e,), N_PAD - 1, jnp.int32)])
    srcp = srcp.reshape(NW * k, CHUNK)
    dstp = dstp.reshape(NW * k, CHUNK)

    kt = 2 * k
    ks = max(8, (kt // 4) // 8 * 8)
    kf = kt - ks
    ksd = max(8, (7 * kt // 20) // 8 * 8)
    kfd = kt - ksd

    degp = _make_deg(ksd, kfd)(dstp)

    grid = (n // BR,)
    y1 = pl.pallas_call(
        _tc1_body,
        grid=grid,
        in_specs=[_degp_spec(),
                  pl.BlockSpec((BR, f_in), lambda i: (i, 0)),
                  _full((f_in, d1))],
        out_specs=pl.BlockSpec((BR, d1), lambda i: (i, 0)),
        out_shape=jax.ShapeDtypeStruct((n, d1), jnp.float32),
    )(degp, x, W1)

    aggp1 = _make_agg(d1, ks, kf)(y1, srcp, dstp)

    y2 = pl.pallas_call(
        _tc2_body,
        grid=grid,
        in_specs=[_degp_spec(), _aggp_spec(d1),
                  pl.BlockSpec((BR, d1), lambda i: (i, 0)),
                  _full((d1, d2)), _full((1, d1))],
        out_specs=pl.BlockSpec((BR, d2), lambda i: (i, 0)),
        out_shape=jax.ShapeDtypeStruct((n, d2), jnp.float32),
    )(degp, aggp1, y1, W2, b1.reshape(1, d1))

    aggp2 = _make_agg(d2, ks, kf)(y2, srcp, dstp)

    out = pl.pallas_call(
        _tc3_body,
        grid=grid,
        in_specs=[_degp_spec(), _aggp_spec(d2),
                  pl.BlockSpec((BR, d2), lambda i: (i, 0)),
                  _full((1, d2)), _full((d2, c)), _full((1, c))],
        out_specs=pl.BlockSpec((BR, c), lambda i: (i, 0)),
        out_shape=jax.ShapeDtypeStruct((n, c), jnp.float32),
    )(degp, aggp2, y2, b2.reshape(1, d2), Wfc, bfc.reshape(1, c))

    return out


# speedup vs baseline: 1.8981x; 1.8981x over previous
import functools

import jax
import jax.numpy as jnp
from jax import lax
from jax.experimental import pallas as pl
from jax.experimental.pallas import tpu as pltpu
from jax.experimental.pallas import tpu_sc as plsc

NC, NS = 2, 16
NW = NC * NS
CHUNK = 128
ROWS_PER_TILE = 626
N_PAD = NS * ROWS_PER_TILE
DEG_W = 16


def _zero_acc_slice(zbuf, acc, sid, d, sem):
    zero = jnp.zeros((16,), jnp.float32)

    def zfill(i, _):
        zbuf[i // (d // 16), pl.ds((i % (d // 16)) * 16, 16)] = zero
        return 0

    lax.fori_loop(0, 16 * (d // 16), zfill, 0)
    base = sid * ROWS_PER_TILE
    nz = ROWS_PER_TILE // 16
    rem = ROWS_PER_TILE - nz * 16
    descs = [pltpu.async_copy(zbuf, acc.at[pl.ds(base + r * 16, 16)], sem)
             for r in range(nz)]
    if rem:
        descs.append(pltpu.async_copy(zbuf.at[pl.ds(0, rem)],
                                      acc.at[pl.ds(base + nz * 16, rem)], sem))
    for dsc in descs:
        dsc.wait()


@functools.lru_cache(maxsize=None)
def _make_agg(d, k, n):
    mesh = plsc.VectorSubcoreMesh(core_axis_name="c", subcore_axis_name="s")

    def body(tbl, srcp, dstp, out, sidx, didx, rows, acc, tbs, semg, sems):
        cid = lax.axis_index("c")
        sid = lax.axis_index("s")
        w = sid * NC + cid
        pltpu.sync_copy(srcp.at[pl.ds(w * k, k)], sidx)
        pltpu.sync_copy(dstp.at[pl.ds(w * k, k)], didx)

        zero = jnp.zeros((16,), jnp.float32)

        def zfill(i, _):
            rows[0, i // (d // 16), pl.ds((i % (d // 16)) * 16, 16)] = zero
            return 0

        lax.fori_loop(0, CHUNK * (d // 16), zfill, 0)
        base = sid * ROWS_PER_TILE
        nfull = ROWS_PER_TILE // CHUNK
        rem = ROWS_PER_TILE - nfull * CHUNK
        zd = [pltpu.async_copy(rows.at[0],
                               acc.at[pl.ds(base + r * CHUNK, CHUNK)], sems)
              for r in range(nfull)]
        if rem:
            zd.append(pltpu.async_copy(
                rows.at[0, pl.ds(0, rem)],
                acc.at[pl.ds(base + nfull * CHUNK, rem)], sems))
        @pl.when(sid == NS - 1)
        def _():
            lr = n - (NS - 1) * ROWS_PER_TILE
            pltpu.async_copy(tbl.at[pl.ds((NS - 1) * ROWS_PER_TILE, lr)],
                             tbs.at[pl.ds((NS - 1) * ROWS_PER_TILE, lr)],
                             semg).wait()

        @pl.when(sid != NS - 1)
        def _():
            pltpu.async_copy(tbl.at[pl.ds(base, ROWS_PER_TILE)],
                             tbs.at[pl.ds(base, ROWS_PER_TILE)], semg).wait()

        for dsc in zd:
            dsc.wait()
        plsc.subcore_barrier()

        pltpu.async_copy(tbs.at[sidx.at[0]], rows.at[0], semg)
        pltpu.async_copy(tbs.at[sidx.at[1]], rows.at[1], semg)

        def step(j, _):
            b = j % 3
            pltpu.make_async_copy(tbs.at[sidx.at[j]], rows.at[b], semg).wait()

            @pl.when(j + 2 < k)
            def _():
                pltpu.async_copy(tbs.at[sidx.at[j + 2]], rows.at[(j + 2) % 3],
                                 semg)

            pltpu.sync_copy(rows.at[b], acc.at[didx.at[j]], add=True)
            return 0

        lax.fori_loop(0, k, step, 0)
        plsc.subcore_barrier()
        pltpu.sync_copy(acc.at[pl.ds(base, ROWS_PER_TILE)],
                        out.at[cid, pl.ds(base, ROWS_PER_TILE)])

    return pl.kernel(
        body,
        out_type=jax.ShapeDtypeStruct((NC, N_PAD, d), jnp.float32),
        mesh=mesh,
        scratch_types=[
            pltpu.VMEM((k, CHUNK), jnp.int32),
            pltpu.VMEM((k, CHUNK), jnp.int32),
            pltpu.VMEM((3, CHUNK, d), jnp.float32),
            pltpu.VMEM_SHARED((N_PAD, d), jnp.float32),
            pltpu.VMEM_SHARED((N_PAD, d), jnp.float32),
            pltpu.SemaphoreType.DMA,
            pltpu.SemaphoreType.DMA,
        ],
        compiler_params=pltpu.CompilerParams(use_tc_tiling_on_sc=False),
    )


@functools.lru_cache(maxsize=None)
def _make_deg(k):
    mesh = plsc.VectorSubcoreMesh(core_axis_name="c", subcore_axis_name="s")

    GB = 8
    assert k % GB == 0

    def body(dstp, out, didx, ones_rows, zbuf, acc, sems):
        cid = lax.axis_index("c")
        sid = lax.axis_index("s")
        w = sid * NC + cid
        pltpu.sync_copy(dstp.at[pl.ds(w * k, k)], didx)
        one = jnp.full((16,), 1.0, jnp.float32)

        def ofill(r, _):
            ones_rows[r, pl.ds(0, 16)] = one
            return 0

        lax.fori_loop(0, CHUNK, ofill, 0)
        _zero_acc_slice(zbuf, acc, sid, DEG_W, sems)
        plsc.subcore_barrier()

        def step(g, _):
            j0 = GB * g
            sc = [pltpu.async_copy(ones_rows, acc.at[didx.at[j0 + i]],
                                   sems, add=True) for i in range(GB)]
            for dsc in sc:
                dsc.wait()
            return 0

        lax.fori_loop(0, k // GB, step, 0)
        plsc.subcore_barrier()
        base = sid * ROWS_PER_TILE
        pltpu.sync_copy(acc.at[pl.ds(base, ROWS_PER_TILE)],
                        out.at[cid, pl.ds(base, ROWS_PER_TILE)])

    return pl.kernel(
        body,
        out_type=jax.ShapeDtypeStruct((NC, N_PAD, DEG_W), jnp.float32),
        mesh=mesh,
        scratch_types=[
            pltpu.VMEM((k, CHUNK), jnp.int32),
            pltpu.VMEM((CHUNK, DEG_W), jnp.float32),
            pltpu.VMEM((16, DEG_W), jnp.float32),
            pltpu.VMEM_SHARED((N_PAD, DEG_W), jnp.float32),
            pltpu.SemaphoreType.DMA,
        ],
        compiler_params=pltpu.CompilerParams(use_tc_tiling_on_sc=False),
    )


BR = 400


def _dis_block(degp):
    p = degp[0, :, :1] + degp[1, :, :1]
    return lax.rsqrt(p + 1.0)


def _tc1_body(degp, x, w1, y1):
    dis = _dis_block(degp)
    y1[...] = dis * jnp.dot(x[...], w1[...], preferred_element_type=jnp.float32)


def _tc2_body(degp, aggp, y1, w2, b1, y2):
    dis = _dis_block(degp)
    h = jnp.maximum(dis * (aggp[0] + aggp[1] + y1[...]) + b1[...], 0.0)
    y2[...] = dis * jnp.dot(h, w2[...], preferred_element_type=jnp.float32)


def _tc3_body(degp, aggp, y2, b2, wfc, bfc, out):
    dis = _dis_block(degp)
    h = jnp.maximum(dis * (aggp[0] + aggp[1] + y2[...]) + b2[...], 0.0)
    logit = jnp.dot(h, wfc[...], preferred_element_type=jnp.float32) + bfc[...]
    m = jnp.max(logit, axis=1, keepdims=True)
    s = jnp.sum(jnp.exp(logit - m), axis=1, keepdims=True)
    out[...] = (logit - m) - jnp.log(s)


def _degp_spec():
    return pl.BlockSpec((NC, BR, DEG_W), lambda i: (0, i, 0))


def _aggp_spec(d):
    return pl.BlockSpec((NC, BR, d), lambda i: (0, i, 0))


def _full(shape):
    nd = len(shape)
    return pl.BlockSpec(shape, lambda i: (0,) * nd)


def kernel(x, edge_index, W1, b1, W2, b2, Wfc, bfc):
    n = x.shape[0]
    e = edge_index.shape[1]
    f_in, d1 = W1.shape
    d2 = W2.shape[1]
    c = Wfc.shape[1]
    assert n % BR == 0

    src = edge_index[0].astype(jnp.int32)
    dst = edge_index[1].astype(jnp.int32)
    k = -(-e // (NW * CHUNK))
    k = -(-k // 8) * 8
    e_pad = NW * k * CHUNK
    srcp = jnp.concatenate([src, jnp.zeros((e_pad - e,), jnp.int32)])
    dstp = jnp.concatenate([dst, jnp.full((e_pad - e,), N_PAD - 1, jnp.int32)])
    srcp = srcp.reshape(NW * k, CHUNK)
    dstp = dstp.reshape(NW * k, CHUNK)

    degp = _make_deg(k)(dstp)

    grid = (n // BR,)
    y1 = pl.pallas_call(
        _tc1_body,
        grid=grid,
        in_specs=[_degp_spec(),
                  pl.BlockSpec((BR, f_in), lambda i: (i, 0)),
                  _full((f_in, d1))],
        out_specs=pl.BlockSpec((BR, d1), lambda i: (i, 0)),
        out_shape=jax.ShapeDtypeStruct((n, d1), jnp.float32),
    )(degp, x, W1)

    aggp1 = _make_agg(d1, k, n)(y1, srcp, dstp)

    y2 = pl.pallas_call(
        _tc2_body,
        grid=grid,
        in_specs=[_degp_spec(), _aggp_spec(d1),
                  pl.BlockSpec((BR, d1), lambda i: (i, 0)),
                  _full((d1, d2)), _full((1, d1))],
        out_specs=pl.BlockSpec((BR, d2), lambda i: (i, 0)),
        out_shape=jax.ShapeDtypeStruct((n, d2), jnp.float32),
    )(degp, aggp1, y1, W2, b1.reshape(1, d1))

    aggp2 = _make_agg(d2, k, n)(y2, srcp, dstp)

    out = pl.pallas_call(
        _tc3_body,
        grid=grid,
        in_specs=[_degp_spec(), _aggp_spec(d2),
                  pl.BlockSpec((BR, d2), lambda i: (i, 0)),
                  _full((1, d2)), _full((d2, c)), _full((1, c))],
        out_specs=pl.BlockSpec((BR, c), lambda i: (i, 0)),
        out_shape=jax.ShapeDtypeStruct((n, c), jnp.float32),
    )(degp, aggp2, y2, b2.reshape(1, d2), Wfc, bfc.reshape(1, c))

    return out


# speedup vs baseline: 2.1246x; 1.1193x over previous
import functools

import jax
import jax.numpy as jnp
from jax import lax
from jax.experimental import pallas as pl
from jax.experimental.pallas import tpu as pltpu
from jax.experimental.pallas import tpu_sc as plsc

NC, NS = 2, 16
NW = NC * NS
CHUNK = 128
ROWS_PER_TILE = 626
N_PAD = NS * ROWS_PER_TILE
DEG_W = 16


def _zero_acc_slice(zbuf, acc, sid, d, sem):
    zero = jnp.zeros((16,), jnp.float32)

    def zfill(i, _):
        zbuf[i // (d // 16), pl.ds((i % (d // 16)) * 16, 16)] = zero
        return 0

    lax.fori_loop(0, 16 * (d // 16), zfill, 0)
    base = sid * ROWS_PER_TILE
    nz = ROWS_PER_TILE // 16
    rem = ROWS_PER_TILE - nz * 16
    descs = [pltpu.async_copy(zbuf, acc.at[pl.ds(base + r * 16, 16)], sem)
             for r in range(nz)]
    if rem:
        descs.append(pltpu.async_copy(zbuf.at[pl.ds(0, rem)],
                                      acc.at[pl.ds(base + nz * 16, rem)], sem))
    for dsc in descs:
        dsc.wait()


@functools.lru_cache(maxsize=None)
def _make_agg(d, k, n):
    mesh = plsc.VectorSubcoreMesh(core_axis_name="c", subcore_axis_name="s")

    def body(tbl, srcp, dstp, out, sidx, didx, rows, acc, tbs, semg, sems):
        cid = lax.axis_index("c")
        sid = lax.axis_index("s")
        w = sid * NC + cid
        pltpu.sync_copy(srcp.at[pl.ds(w * k, k)], sidx)
        pltpu.sync_copy(dstp.at[pl.ds(w * k, k)], didx)

        zero = jnp.zeros((16,), jnp.float32)

        def zfill(i, _):
            rows[0, i // (d // 16), pl.ds((i % (d // 16)) * 16, 16)] = zero
            return 0

        lax.fori_loop(0, CHUNK * (d // 16), zfill, 0)
        base = sid * ROWS_PER_TILE
        nfull = ROWS_PER_TILE // CHUNK
        rem = ROWS_PER_TILE - nfull * CHUNK
        zd = [pltpu.async_copy(rows.at[0],
                               acc.at[pl.ds(base + r * CHUNK, CHUNK)], sems)
              for r in range(nfull)]
        if rem:
            zd.append(pltpu.async_copy(
                rows.at[0, pl.ds(0, rem)],
                acc.at[pl.ds(base + nfull * CHUNK, rem)], sems))
        @pl.when(sid == NS - 1)
        def _():
            lr = n - (NS - 1) * ROWS_PER_TILE
            pltpu.async_copy(tbl.at[pl.ds((NS - 1) * ROWS_PER_TILE, lr)],
                             tbs.at[pl.ds((NS - 1) * ROWS_PER_TILE, lr)],
                             semg).wait()

        @pl.when(sid != NS - 1)
        def _():
            pltpu.async_copy(tbl.at[pl.ds(base, ROWS_PER_TILE)],
                             tbs.at[pl.ds(base, ROWS_PER_TILE)], semg).wait()

        for dsc in zd:
            dsc.wait()
        plsc.subcore_barrier()

        pltpu.async_copy(tbs.at[sidx.at[0]], rows.at[0], semg)
        pltpu.async_copy(tbs.at[sidx.at[1]], rows.at[1], semg)

        def step(j, _):
            b = j % 3
            pltpu.make_async_copy(tbs.at[sidx.at[j]], rows.at[b], semg).wait()

            @pl.when(j + 2 < k)
            def _():
                pltpu.async_copy(tbs.at[sidx.at[j + 2]], rows.at[(j + 2) % 3],
                                 semg)

            pltpu.sync_copy(rows.at[b], acc.at[didx.at[j]], add=True)
            return 0

        lax.fori_loop(0, k, step, 0)
        plsc.subcore_barrier()
        pltpu.sync_copy(acc.at[pl.ds(base, ROWS_PER_TILE)],
                        out.at[cid, pl.ds(base, ROWS_PER_TILE)])

    return pl.kernel(
        body,
        out_type=jax.ShapeDtypeStruct((NC, N_PAD, d), jnp.float32),
        mesh=mesh,
        scratch_types=[
            pltpu.VMEM((k, CHUNK), jnp.int32),
            pltpu.VMEM((k, CHUNK), jnp.int32),
            pltpu.VMEM((3, CHUNK, d), jnp.float32),
            pltpu.VMEM_SHARED((N_PAD, d), jnp.float32),
            pltpu.VMEM_SHARED((N_PAD, d), jnp.float32),
            pltpu.SemaphoreType.DMA,
            pltpu.SemaphoreType.DMA,
        ],
        compiler_params=pltpu.CompilerParams(use_tc_tiling_on_sc=False),
    )


@functools.lru_cache(maxsize=None)
def _make_deg(k):
    mesh = plsc.VectorSubcoreMesh(core_axis_name="c", subcore_axis_name="s")

    GB = 8
    assert k % GB == 0

    def body(dstp, out, didx, ones_rows, zbuf, acc, sems):
        cid = lax.axis_index("c")
        sid = lax.axis_index("s")
        w = sid * NC + cid
        pltpu.sync_copy(dstp.at[pl.ds(w * k, k)], didx)
        one = jnp.full((16,), 1.0, jnp.float32)

        def ofill(r, _):
            ones_rows[r, pl.ds(0, 16)] = one
            return 0

        lax.fori_loop(0, CHUNK, ofill, 0)
        _zero_acc_slice(zbuf, acc, sid, DEG_W, sems)
        plsc.subcore_barrier()

        def step(g, _):
            j0 = GB * g
            sc = [pltpu.async_copy(ones_rows, acc.at[didx.at[j0 + i]],
                                   sems, add=True) for i in range(GB)]
            for dsc in sc:
                dsc.wait()
            return 0

        lax.fori_loop(0, k // GB, step, 0)
        plsc.subcore_barrier()
        base = sid * ROWS_PER_TILE
        pltpu.sync_copy(acc.at[pl.ds(base, ROWS_PER_TILE), pl.ds(0, 8)],
                        out.at[cid, pl.ds(base, ROWS_PER_TILE)])

    return pl.kernel(
        body,
        out_type=jax.ShapeDtypeStruct((NC, N_PAD, 8), jnp.float32),
        mesh=mesh,
        scratch_types=[
            pltpu.VMEM((k, CHUNK), jnp.int32),
            pltpu.VMEM((CHUNK, DEG_W), jnp.float32),
            pltpu.VMEM((16, DEG_W), jnp.float32),
            pltpu.VMEM_SHARED((N_PAD, DEG_W), jnp.float32),
            pltpu.SemaphoreType.DMA,
        ],
        compiler_params=pltpu.CompilerParams(use_tc_tiling_on_sc=False),
    )


BR = 2000


def _dis_block(degp):
    p = degp[0, :, :1] + degp[1, :, :1]
    return lax.rsqrt(p + 1.0)


def _tc1_body(degp, x, w1, y1):
    dis = _dis_block(degp)
    y1[...] = dis * jnp.dot(x[...], w1[...], preferred_element_type=jnp.float32)


def _tc2_body(degp, aggp, y1, w2, b1, y2):
    dis = _dis_block(degp)
    h = jnp.maximum(dis * (aggp[0] + aggp[1] + y1[...]) + b1[...], 0.0)
    y2[...] = dis * jnp.dot(h, w2[...], preferred_element_type=jnp.float32)


def _tc3_body(degp, aggp, y2, b2, wfc, bfc, out):
    dis = _dis_block(degp)
    h = jnp.maximum(dis * (aggp[0] + aggp[1] + y2[...]) + b2[...], 0.0)
    logit = jnp.dot(h, wfc[...], preferred_element_type=jnp.float32) + bfc[...]
    m = jnp.max(logit, axis=1, keepdims=True)
    s = jnp.sum(jnp.exp(logit - m), axis=1, keepdims=True)
    out[...] = (logit - m) - jnp.log(s)


def _degp_spec():
    return pl.BlockSpec((NC, BR, 8), lambda i: (0, i, 0))


def _aggp_spec(d):
    return pl.BlockSpec((NC, BR, d), lambda i: (0, i, 0))


def _full(shape):
    nd = len(shape)
    return pl.BlockSpec(shape, lambda i: (0,) * nd)


def kernel(x, edge_index, W1, b1, W2, b2, Wfc, bfc):
    n = x.shape[0]
    e = edge_index.shape[1]
    f_in, d1 = W1.shape
    d2 = W2.shape[1]
    c = Wfc.shape[1]
    assert n % BR == 0

    src = edge_index[0].astype(jnp.int32)
    dst = edge_index[1].astype(jnp.int32)
    k = -(-e // (NW * CHUNK))
    k = -(-k // 8) * 8
    e_pad = NW * k * CHUNK
    srcp = jnp.concatenate([src, jnp.zeros((e_pad - e,), jnp.int32)])
    dstp = jnp.concatenate([dst, jnp.full((e_pad - e,), N_PAD - 1, jnp.int32)])
    srcp = srcp.reshape(NW * k, CHUNK)
    dstp = dstp.reshape(NW * k, CHUNK)

    degp = _make_deg(k)(dstp)

    grid = (n // BR,)
    y1 = pl.pallas_call(
        _tc1_body,
        grid=grid,
        in_specs=[_degp_spec(),
                  pl.BlockSpec((BR, f_in), lambda i: (i, 0)),
                  _full((f_in, d1))],
        out_specs=pl.BlockSpec((BR, d1), lambda i: (i, 0)),
        out_shape=jax.ShapeDtypeStruct((n, d1), jnp.float32),
    )(degp, x, W1)

    aggp1 = _make_agg(d1, k, n)(y1, srcp, dstp)

    y2 = pl.pallas_call(
        _tc2_body,
        grid=grid,
        in_specs=[_degp_spec(), _aggp_spec(d1),
                  pl.BlockSpec((BR, d1), lambda i: (i, 0)),
                  _full((d1, d2)), _full((1, d1))],
        out_specs=pl.BlockSpec((BR, d2), lambda i: (i, 0)),
        out_shape=jax.ShapeDtypeStruct((n, d2), jnp.float32),
    )(degp, aggp1, y1, W2, b1.reshape(1, d1))

    aggp2 = _make_agg(d2, k, n)(y2, srcp, dstp)

    out = pl.pallas_call(
        _tc3_body,
        grid=grid,
        in_specs=[_degp_spec(), _aggp_spec(d2),
                  pl.BlockSpec((BR, d2), lambda i: (i, 0)),
                  _full((1, d2)), _full((d2, c)), _full((1, c))],
        out_specs=pl.BlockSpec((BR, c), lambda i: (i, 0)),
        out_shape=jax.ShapeDtypeStruct((n, c), jnp.float32),
    )(degp, aggp2, y2, b2.reshape(1, d2), Wfc, bfc.reshape(1, c))

    return out
